# baseline (speedup 1.0000x reference)
import functools

import jax
import jax.numpy as jnp
from jax import lax
from jax.experimental import pallas as pl
from jax.experimental.pallas import tpu as pltpu
from jax.experimental.pallas import tpu_sc as plsc

L = 16
NC, NS = 2, 16
NW = NC * NS
NSLOT = 4


def kernel(objs, vectors, lengths):
    B, = objs.shape
    N, P, D = vectors.shape
    row = P * D
    vec2d = vectors.reshape(N, row)

    bpw = B // NW
    C = 32
    nch = bpw // C
    LCH = 128

    mesh = plsc.VectorSubcoreMesh(core_axis_name="c", subcore_axis_name="s",
                                  num_cores=NC, num_subcores=NS)

    @functools.partial(
        pl.kernel,
        out_type=jax.ShapeDtypeStruct((D, B), jnp.float32),
        mesh=mesh,
        compiler_params=pltpu.CompilerParams(use_tc_tiling_on_sc=True, needs_layout_passes=False),
        scratch_types=[
            pltpu.VMEM((bpw,), jnp.int32),
            pltpu.VMEM((bpw,), jnp.int32),
            pltpu.VMEM((bpw + L,), jnp.float32),
            [pltpu.VMEM((C, row), jnp.float32) for _ in range(NSLOT)],
            pltpu.VMEM((C, D), jnp.float32),
            pltpu.VMEM((D, bpw), jnp.float32),
            pltpu.SemaphoreType.DMA,
            [pltpu.SemaphoreType.DMA for _ in range(NSLOT)],
        ],
    )
    def sc_kernel(objs_hbm, vec_hbm, len_hbm, out_hbm,
                  idx_v, lens_v, recip_v, bufs, obuf, otbuf,
                  lsem, sems):
        wid = lax.axis_index("s") * NC + lax.axis_index("c")
        base = wid * bpw

        pltpu.sync_copy(objs_hbm.at[pl.ds(base, bpw)], idx_v)

        lens_handles = [
            pltpu.async_copy(len_hbm.at[idx_v.at[pl.ds(k * LCH, LCH)]],
                             lens_v.at[pl.ds(k * LCH, LCH)], lsem)
            for k in range(bpw // LCH)
        ]

        def fire(c, b):
            pltpu.async_copy(
                vec_hbm.at[idx_v.at[pl.ds(c * C, C)]], bufs[b], sems[b])

        for b in range(NSLOT):
            fire(b, b)

        for h in lens_handles:
            h.wait()
        for g in range(bpw // L):
            lv = lens_v[pl.ds(g * L, L)]
            recip_v[pl.ds(g * L, L)] = 1.0 / (lv.astype(jnp.float32) + 1e-5)

        zero_idx = jnp.zeros((L, 1), jnp.int32)
        bcast_dnums = lax.GatherDimensionNumbers(
            offset_dims=(), collapsed_slice_dims=(0,), start_index_map=(0,))

        def bcast0(v):
            return lax.gather(v, zero_idx, bcast_dnums, (1,),
                              mode=lax.GatherScatterMode.PROMISE_IN_BOUNDS)

        lane_iota = lax.iota(jnp.int32, L)

        def body(i, _):
            for b in range(NSLOT):
                c = NSLOT * i + b

                pltpu.make_async_copy(
                    vec_hbm.at[idx_v.at[pl.ds(0, C)]], bufs[b], sems[b]).wait()

                def obj_body(k, _, b=b, c=c):
                    rcp = plsc.load_gather(
                        recip_v, [jnp.full((L,), c * C + k, jnp.int32)])
                    for db in range(D // L):
                        acc = bufs[b][k, pl.ds(db * L, L)]
                        for p in range(1, P):
                            acc = acc + bufs[b][k, pl.ds(p * D + db * L, L)]
                        obuf[k, pl.ds(db * L, L)] = acc * rcp
                    return 0

                lax.fori_loop(0, C, obj_body, 0)

                @pl.when(c + NSLOT < nch)
                def _fire(c=c, b=b):
                    fire(c + NSLOT, b)

                for q in range(C // L):
                    rows = lane_iota + q * L
                    for d in range(D):
                        col = plsc.load_gather(
                            obuf, [rows, jnp.full((L,), d, jnp.int32)])
                        otbuf[d, pl.ds(c * C + q * L, L)] = col
            return 0

        lax.fori_loop(0, nch // NSLOT, body, 0)

        pltpu.sync_copy(otbuf, out_hbm.at[:, pl.ds(base, bpw)])

    return sc_kernel(objs, vec2d, lengths).T


# speedup vs baseline: 1.0082x; 1.0082x over previous
import functools

import jax
import jax.numpy as jnp
from jax import lax
from jax.experimental import pallas as pl
from jax.experimental.pallas import tpu as pltpu
from jax.experimental.pallas import tpu_sc as plsc

L = 16
NC, NS = 2, 16
NW = NC * NS
NSLOT = 4


def kernel(objs, vectors, lengths):
    B, = objs.shape
    N, P, D = vectors.shape
    row = P * D
    vec2d = vectors.reshape(N, row)

    bpw = B // NW
    C = 32
    nch = bpw // C
    LCH = 128

    mesh = plsc.VectorSubcoreMesh(core_axis_name="c", subcore_axis_name="s",
                                  num_cores=NC, num_subcores=NS)

    @functools.partial(
        pl.kernel,
        out_type=jax.ShapeDtypeStruct((D, B), jnp.float32),
        mesh=mesh,
        compiler_params=pltpu.CompilerParams(use_tc_tiling_on_sc=True, needs_layout_passes=False),
        scratch_types=[
            pltpu.VMEM((bpw,), jnp.int32),
            pltpu.VMEM((bpw,), jnp.int32),
            pltpu.VMEM((bpw + L,), jnp.float32),
            [pltpu.VMEM((C, row), jnp.float32) for _ in range(NSLOT)],
            pltpu.VMEM((C, D + 1), jnp.float32),
            pltpu.VMEM((D, bpw), jnp.float32),
            pltpu.SemaphoreType.DMA,
            [pltpu.SemaphoreType.DMA for _ in range(NSLOT)],
        ],
    )
    def sc_kernel(objs_hbm, vec_hbm, len_hbm, out_hbm,
                  idx_v, lens_v, recip_v, bufs, obuf, otbuf,
                  lsem, sems):
        wid = lax.axis_index("s") * NC + lax.axis_index("c")
        base = wid * bpw

        pltpu.sync_copy(objs_hbm.at[pl.ds(base, bpw)], idx_v)

        lens_handles = [
            pltpu.async_copy(len_hbm.at[idx_v.at[pl.ds(k * LCH, LCH)]],
                             lens_v.at[pl.ds(k * LCH, LCH)], lsem)
            for k in range(bpw // LCH)
        ]

        def fire(c, b):
            pltpu.async_copy(
                vec_hbm.at[idx_v.at[pl.ds(c * C, C)]], bufs[b], sems[b])

        for b in range(NSLOT):
            fire(b, b)

        for h in lens_handles:
            h.wait()
        for g in range(bpw // L):
            lv = lens_v[pl.ds(g * L, L)]
            recip_v[pl.ds(g * L, L)] = 1.0 / (lv.astype(jnp.float32) + 1e-5)

        zero_idx = jnp.zeros((L, 1), jnp.int32)
        bcast_dnums = lax.GatherDimensionNumbers(
            offset_dims=(), collapsed_slice_dims=(0,), start_index_map=(0,))

        def bcast0(v):
            return lax.gather(v, zero_idx, bcast_dnums, (1,),
                              mode=lax.GatherScatterMode.PROMISE_IN_BOUNDS)

        lane_iota = lax.iota(jnp.int32, L)

        def body(i, _):
            for b in range(NSLOT):
                c = NSLOT * i + b

                pltpu.make_async_copy(
                    vec_hbm.at[idx_v.at[pl.ds(0, C)]], bufs[b], sems[b]).wait()

                def obj_body(k, _, b=b, c=c):
                    rcp = plsc.load_gather(
                        recip_v, [jnp.full((L,), c * C + k, jnp.int32)])
                    for db in range(D // L):
                        acc = bufs[b][k, pl.ds(db * L, L)]
                        for p in range(1, P):
                            acc = acc + bufs[b][k, pl.ds(p * D + db * L, L)]
                        obuf[k, pl.ds(db * L, L)] = acc * rcp
                    return 0

                lax.fori_loop(0, C, obj_body, 0)

                @pl.when(c + NSLOT < nch)
                def _fire(c=c, b=b):
                    fire(c + NSLOT, b)

                for q in range(C // L):
                    rows = lane_iota + q * L
                    for d in range(D):
                        col = plsc.load_gather(
                            obuf, [rows, jnp.full((L,), d, jnp.int32)])
                        otbuf[d, pl.ds(c * C + q * L, L)] = col
            return 0

        lax.fori_loop(0, nch // NSLOT, body, 0)

        pltpu.sync_copy(otbuf, out_hbm.at[:, pl.ds(base, bpw)])

    return sc_kernel(objs, vec2d, lengths).T


# speedup vs baseline: 1.1102x; 1.1012x over previous
import functools

import jax
import jax.numpy as jnp
from jax import lax
from jax.experimental import pallas as pl
from jax.experimental.pallas import tpu as pltpu
from jax.experimental.pallas import tpu_sc as plsc

L = 16
NC, NS = 2, 16
NW = NC * NS
NSLOT = 4


def kernel(objs, vectors, lengths):
    B, = objs.shape
    N, P, D = vectors.shape
    row = P * D
    vec2d = vectors.reshape(N, row)

    bpw = B // NW
    C = 32
    nch = bpw // C
    LCH = 128

    mesh = plsc.VectorSubcoreMesh(core_axis_name="c", subcore_axis_name="s",
                                  num_cores=NC, num_subcores=NS)

    @functools.partial(
        pl.kernel,
        out_type=jax.ShapeDtypeStruct((B, D), jnp.float32),
        mesh=mesh,
        compiler_params=pltpu.CompilerParams(use_tc_tiling_on_sc=True),
        scratch_types=[
            pltpu.VMEM((bpw,), jnp.int32),
            pltpu.VMEM((bpw,), jnp.int32),
            pltpu.VMEM((bpw + L,), jnp.float32),
            [pltpu.VMEM((C, row), jnp.float32) for _ in range(NSLOT)],
            [pltpu.VMEM((C, D), jnp.float32) for _ in range(NSLOT)],
            pltpu.SemaphoreType.DMA,
            [pltpu.SemaphoreType.DMA for _ in range(NSLOT)],
            [pltpu.SemaphoreType.DMA for _ in range(NSLOT)],
        ],
    )
    def sc_kernel(objs_hbm, vec_hbm, len_hbm, out_hbm,
                  idx_v, lens_v, recip_v, bufs, obufs,
                  lsem, sems, osems):
        wid = lax.axis_index("s") * NC + lax.axis_index("c")
        base = wid * bpw

        pltpu.sync_copy(objs_hbm.at[pl.ds(base, bpw)], idx_v)

        lens_handles = [
            pltpu.async_copy(len_hbm.at[idx_v.at[pl.ds(k * LCH, LCH)]],
                             lens_v.at[pl.ds(k * LCH, LCH)], lsem)
            for k in range(bpw // LCH)
        ]

        def fire(c, b):
            pltpu.async_copy(
                vec_hbm.at[idx_v.at[pl.ds(c * C, C)]], bufs[b], sems[b])

        for b in range(NSLOT):
            fire(b, b)

        for h in lens_handles:
            h.wait()
        for g in range(bpw // L):
            lv = lens_v[pl.ds(g * L, L)]
            recip_v[pl.ds(g * L, L)] = 1.0 / (lv.astype(jnp.float32) + 1e-5)

        zero_idx = jnp.zeros((L, 1), jnp.int32)
        bcast_dnums = lax.GatherDimensionNumbers(
            offset_dims=(), collapsed_slice_dims=(0,), start_index_map=(0,))

        def bcast0(v):
            return lax.gather(v, zero_idx, bcast_dnums, (1,),
                              mode=lax.GatherScatterMode.PROMISE_IN_BOUNDS)

        def out_wait(b):
            pltpu.make_async_copy(
                obufs[b], out_hbm.at[pl.ds(base, C)], osems[b]).wait()

        def body(i, _):
            for b in range(NSLOT):
                c = NSLOT * i + b

                @pl.when(c >= NSLOT)
                def _drain(b=b):
                    out_wait(b)

                pltpu.make_async_copy(
                    vec_hbm.at[idx_v.at[pl.ds(0, C)]], bufs[b], sems[b]).wait()

                def obj_body(k, _, b=b, c=c):
                    rcp = bcast0(recip_v[pl.ds(c * C + k, L)])
                    for db in range(D // L):
                        acc = bufs[b][k, pl.ds(db * L, L)]
                        for p in range(1, P):
                            acc = acc + bufs[b][k, pl.ds(p * D + db * L, L)]
                        obufs[b][k, pl.ds(db * L, L)] = acc * rcp
                    return 0

                lax.fori_loop(0, C, obj_body, 0)

                @pl.when(c + NSLOT < nch)
                def _fire(c=c, b=b):
                    fire(c + NSLOT, b)

                pltpu.async_copy(
                    obufs[b], out_hbm.at[pl.ds(base + c * C, C)], osems[b])
            return 0

        lax.fori_loop(0, nch // NSLOT, body, 0)

        for b in range(NSLOT):
            out_wait(b)

    return sc_kernel(objs, vec2d, lengths)
